# Initial kernel scaffold; baseline (speedup 1.0000x reference)
import jax
import jax.numpy as jnp
from jax.experimental import pallas as pl


def kernel(o, m, c, edge_index_o, edge_index_m, edge_index_c, W1_rel, W1_root, b1, W2_rel, W2_root, b2, W3_rel, W3_root, b3, W4_rel, W4_root, b4):
    raise NotImplementedError("write your pallas kernel here")


# speedup vs baseline: 57.6869x; 57.6869x over previous
import jax
import jax.numpy as jnp
from jax import lax
from jax.experimental import pallas as pl
from jax.experimental.pallas import tpu as pltpu
from jax.experimental.pallas import tpu_sc as plsc

N = 100000
E = 3200000
NCH = E // 128
KM = 8
NMAC = NCH // KM
NW = 32
NV = N // 16
VPW = 196
RT = 6256
RTL = N - 15 * RT
F = 8

assert NCH * 128 == E and NMAC * KM == NCH and NV * 16 == N
assert (NW - 1) * NV // NW + VPW == NV

_MESH = plsc.VectorSubcoreMesh(core_axis_name="c", subcore_axis_name="s")
_SEG_PARAMS = pltpu.CompilerParams(use_tc_tiling_on_sc=False)
_DENSE_PARAMS = pltpu.CompilerParams(use_tc_tiling_on_sc=False,
                                     needs_layout_passes=False)
_NEG = -3.4e38


def _iota16():
    return lax.broadcasted_iota(jnp.int32, (16,), 0)


def _scalars(wv, n):
    vecs = [wv[pl.ds(16 * t, 16)] for t in range((n + 15) // 16)]
    return [vecs[k // 16][k % 16] for k in range(n)]


def _seg_job(tab, srch, dsth, acc, rows, si, di, isem, gsem, ssem, mb, nm):
    def idx_pair(mm, q):
        return (pltpu.make_async_copy(srch.at[pl.ds((mb + mm) * KM, KM)],
                                      si.at[q], isem[q]),
                pltpu.make_async_copy(dsth.at[pl.ds((mb + mm) * KM, KM)],
                                      di.at[q], isem[q]))

    def scat_wait(b, q):
        for j in range(KM):
            pltpu.make_async_copy(rows.at[b, j], acc.at[di.at[q, j]],
                                  ssem[b]).wait()

    for k in (0, 1):
        a, bb = idx_pair(k, k)
        a.start()
        bb.start()

    def body(m, q, b):
        @pl.when(m < nm)
        def _():
            a, bb = idx_pair(m, q)
            a.wait()
            bb.wait()

            @pl.when(m >= 2)
            def _():
                scat_wait(b, (q - 2) % 4)

            @pl.when(m + 2 < nm)
            def _():
                a2, b2 = idx_pair(m + 2, (q + 2) % 4)
                a2.start()
                b2.start()

            gd = [pltpu.make_async_copy(tab.at[si.at[q, j]],
                                        rows.at[b, j], gsem[b])
                  for j in range(KM)]
            for g in gd:
                g.start()
            for g in gd:
                g.wait()
            for j in range(KM):
                pltpu.async_copy(rows.at[b, j], acc.at[di.at[q, j]],
                                 ssem[b], add=True)

    def quad(qi, carry):
        for jj in range(4):
            body(qi * 4 + jj, jj, jj & 1)
        return carry

    lax.fori_loop(0, 25, quad, 0)

    @pl.when(nm == 97)
    def _():
        scat_wait(0, 0)
        scat_wait(1, 3)

    @pl.when(nm == 98)
    def _():
        scat_wait(1, 1)
        scat_wait(0, 0)


def _make_segsum(nj):
    def body(*args):
        tabs = args[0:nj]
        srcs = args[nj:2 * nj]
        dsts = args[2 * nj:3 * nj]
        z = args[3 * nj]
        outs = args[3 * nj + 1:4 * nj + 1]
        scr = args[4 * nj + 1:]
        si, di, rows = scr[0], scr[1], scr[2]
        accs = scr[3:3 + nj]
        sems = scr[3 + nj:]
        isem, gsem, ssem = sems[0:4], sems[4:6], sems[6:8]

        c = lax.axis_index("c")
        s = lax.axis_index("s")
        w = c * 16 + s

        for j in range(nj):
            aj = accs[j]

            @pl.when(s < 15)
            def _(aj=aj):
                pltpu.sync_copy(z, aj.at[pl.ds(s * RT, RT)])

            @pl.when(s == 15)
            def _(aj=aj):
                pltpu.sync_copy(z.at[pl.ds(0, RTL)], aj.at[pl.ds(15 * RT, RTL)])

        plsc.subcore_barrier()

        mb = (w * NMAC) // NW
        me = ((w + 1) * NMAC) // NW
        nm = me - mb
        for j in range(nj):
            _seg_job(tabs[j], srcs[j], dsts[j], accs[j], rows,
                     si, di, isem, gsem, ssem, mb, nm)

        plsc.subcore_barrier()

        for j in range(nj):
            aj = accs[j]
            oj = outs[j]

            @pl.when(s < 15)
            def _(aj=aj, oj=oj):
                pltpu.sync_copy(aj.at[pl.ds(s * RT, RT)],
                                oj.at[c, pl.ds(s * RT, RT)])

            @pl.when(s == 15)
            def _(aj=aj, oj=oj):
                pltpu.sync_copy(aj.at[pl.ds(15 * RT, RTL)],
                                oj.at[c, pl.ds(15 * RT, RTL)])

    out_type = [jax.ShapeDtypeStruct((2, N, F), jnp.float32)
                for _ in range(nj)]
    scratch = [
        pltpu.VMEM((4, KM, 128), jnp.int32),
        pltpu.VMEM((4, KM, 128), jnp.int32),
        pltpu.VMEM((2, KM, 128, F), jnp.float32),
    ]
    scratch += [pltpu.VMEM_SHARED((N, F), jnp.float32) for _ in range(nj)]
    scratch += [pltpu.SemaphoreType.DMA] * 8
    return pl.kernel(body, out_type=out_type, mesh=_MESH,
                     compiler_params=_SEG_PARAMS, scratch_types=scratch)


_seg2 = _make_segsum(2)
_seg1 = _make_segsum(1)


def _phase_b_body(o_f, m_f, c_f, p1o, p1m, p3c, wpk,
                  h1o, h1m, g0o, r0o,
                  xo, xm, xc, po, pmb, pcb, ho, hm, gb, rb, wv):
    c = lax.axis_index("c")
    s = lax.axis_index("s")
    w = c * 16 + s
    vb = (w * NV) // NW
    pltpu.sync_copy(wpk, wv)
    W = _scalars(wv, 184)
    iota = _iota16()
    zero = jnp.zeros((16,), jnp.float32)

    for sub in range(2):
        nb0 = vb * 16 + sub * 1568
        pltpu.sync_copy(o_f.at[pl.ds(nb0 * 2, 3136)], xo)
        pltpu.sync_copy(m_f.at[pl.ds(nb0 * 2, 3136)], xm)
        pltpu.sync_copy(c_f.at[pl.ds(nb0 * 2, 3136)], xc)
        for h in range(2):
            pltpu.sync_copy(p1o.at[pl.ds(h * (N * 8) + nb0 * 8, 12544)],
                            po.at[pl.ds(h * 12544, 12544)])
            pltpu.sync_copy(p1m.at[pl.ds(h * (N * 8) + nb0 * 8, 12544)],
                            pmb.at[pl.ds(h * 12544, 12544)])
            pltpu.sync_copy(p3c.at[pl.ds(h * (N * 8) + nb0 * 8, 12544)],
                            pcb.at[pl.ds(h * 12544, 12544)])

        def vloop(i, carry):
            nid = i * 16 + iota
            nid2 = nid * 2
            nid8 = nid * 8

            for xb, pb, hb in ((xo, po, ho), (xm, pmb, hm)):
                x0 = plsc.load_gather(xb, [nid2])
                x1 = plsc.load_gather(xb, [nid2 + 1])
                a0 = (plsc.load_gather(pb, [nid8])
                      + plsc.load_gather(pb, [nid8 + 12544]))
                a1 = (plsc.load_gather(pb, [nid8 + 1])
                      + plsc.load_gather(pb, [nid8 + 12545]))
                for jf in range(8):
                    hv = (a0 * W[jf] + a1 * W[8 + jf]
                          + x0 * W[16 + jf] + x1 * W[24 + jf] + W[32 + jf])
                    hv = jnp.where(hv > 0, hv, jnp.exp(hv) - 1.0)
                    plsc.store_scatter(hb, [nid8 + jf], hv)

            c0 = plsc.load_gather(xc, [nid2])
            c1 = plsc.load_gather(xc, [nid2 + 1])
            a0 = (plsc.load_gather(pcb, [nid8])
                  + plsc.load_gather(pcb, [nid8 + 12544]))
            a1 = (plsc.load_gather(pcb, [nid8 + 1])
                  + plsc.load_gather(pcb, [nid8 + 12545]))
            g0v = zero
            g1v = zero
            r0v = zero
            r1v = zero
            for jf in range(16):
                hv = (a0 * W[40 + jf] + a1 * W[56 + jf]
                      + c0 * W[72 + jf] + c1 * W[88 + jf] + W[104 + jf])
                hv = jnp.where(hv > 0, hv, jnp.exp(hv) - 1.0)
                g0v = g0v + hv * W[120 + jf * 2]
                g1v = g1v + hv * W[121 + jf * 2]
                r0v = r0v + hv * W[152 + jf * 2]
                r1v = r1v + hv * W[153 + jf * 2]
            plsc.store_scatter(gb, [nid8], g0v)
            plsc.store_scatter(gb, [nid8 + 1], g1v)
            for jf in range(2, 8):
                plsc.store_scatter(gb, [nid8 + jf], zero)
            plsc.store_scatter(rb, [nid2], r0v)
            plsc.store_scatter(rb, [nid2 + 1], r1v)
            return carry

        lax.fori_loop(0, 98, vloop, 0)

        pltpu.sync_copy(ho, h1o.at[pl.ds(nb0 * 8, 12544)])
        pltpu.sync_copy(hm, h1m.at[pl.ds(nb0 * 8, 12544)])
        pltpu.sync_copy(gb, g0o.at[pl.ds(nb0 * 8, 12544)])
        pltpu.sync_copy(rb, r0o.at[pl.ds(nb0 * 2, 3136)])


_phase_b = pl.kernel(
    _phase_b_body,
    out_type=[jax.ShapeDtypeStruct((N * 8,), jnp.float32),
              jax.ShapeDtypeStruct((N * 8,), jnp.float32),
              jax.ShapeDtypeStruct((N * 8,), jnp.float32),
              jax.ShapeDtypeStruct((N * 2,), jnp.float32)],
    mesh=_MESH,
    compiler_params=_DENSE_PARAMS,
    scratch_types=[pltpu.VMEM((3136,), jnp.float32),
                   pltpu.VMEM((3136,), jnp.float32),
                   pltpu.VMEM((3136,), jnp.float32),
                   pltpu.VMEM((25088,), jnp.float32),
                   pltpu.VMEM((25088,), jnp.float32),
                   pltpu.VMEM((25088,), jnp.float32),
                   pltpu.VMEM((12544,), jnp.float32),
                   pltpu.VMEM((12544,), jnp.float32),
                   pltpu.VMEM((12544,), jnp.float32),
                   pltpu.VMEM((3136,), jnp.float32),
                   pltpu.VMEM((192,), jnp.float32)])


def _phase_d1_body(p2o, p2m, h1o, h1m, wpk, pmo, pmm, pb, hb, ob, wv):
    c = lax.axis_index("c")
    s = lax.axis_index("s")
    w = c * 16 + s
    vb = (w * NV) // NW
    pltpu.sync_copy(wpk, wv)
    W = _scalars(wv, 272)
    iota = _iota16()

    for gi, (p2, h1) in enumerate(((p2o, h1o), (p2m, h1m))):
        for h in range(2):
            pltpu.sync_copy(p2.at[pl.ds(h * (N * 8) + vb * 128, 25088)],
                            pb.at[pl.ds(h * 25088, 25088)])
        pltpu.sync_copy(h1.at[pl.ds(vb * 128, 25088)], hb)

        def vloop(i, mx):
            nid8 = (i * 16 + iota) * 8
            a = [plsc.load_gather(pb, [nid8 + k])
                 + plsc.load_gather(pb, [nid8 + 25088 + k]) for k in range(8)]
            hv = [plsc.load_gather(hb, [nid8 + k]) for k in range(8)]
            out = []
            for j in range(16):
                v = a[0] * W[j]
                for k in range(1, 8):
                    v = v + a[k] * W[k * 16 + j]
                for k in range(8):
                    v = v + hv[k] * W[128 + k * 16 + j]
                v = v + W[256 + j]
                out.append(jnp.maximum(mx[j], v))
            return tuple(out)

        init = tuple(jnp.full((16,), _NEG, jnp.float32) for _ in range(16))
        mx = lax.fori_loop(0, VPW, vloop, init)
        for j in range(16):
            ob[pl.ds(gi * 256 + j * 16, 16)] = mx[j]

    pltpu.sync_copy(ob.at[pl.ds(0, 256)], pmo.at[w])
    pltpu.sync_copy(ob.at[pl.ds(256, 256)], pmm.at[w])


_phase_d1 = pl.kernel(
    _phase_d1_body,
    out_type=[jax.ShapeDtypeStruct((NW, 256), jnp.float32),
              jax.ShapeDtypeStruct((NW, 256), jnp.float32)],
    mesh=_MESH,
    compiler_params=_DENSE_PARAMS,
    scratch_types=[pltpu.VMEM((50176,), jnp.float32),
                   pltpu.VMEM((25088,), jnp.float32),
                   pltpu.VMEM((512,), jnp.float32),
                   pltpu.VMEM((272,), jnp.float32)])


def _phase_d2_body(pmo, pmm, p4, p3c, r0, c_f, wpk, outf,
                   pmbo, pmbm, p4b, p3b, rbuf, cbuf, ob, wv):
    c = lax.axis_index("c")
    s = lax.axis_index("s")
    w = c * 16 + s
    vb = (w * NV) // NW
    pltpu.sync_copy(wpk, wv)
    pltpu.sync_copy(pmo, pmbo)
    pltpu.sync_copy(pmm, pmbm)

    feats = []
    for pmb in (pmbo, pmbm):
        for j in range(16):
            v = pmb[pl.ds(j * 16, 16)]
            for ww in range(1, NW):
                v = jnp.maximum(v, pmb[pl.ds(ww * 256 + j * 16, 16)])
            feats.append(jnp.max(v))

    W = _scalars(wv, 130)
    pc = []
    pr = []
    for k in range(2):
        acc_c = feats[0] * W[k]
        acc_r = feats[0] * W[64 + k]
        for j in range(1, 32):
            acc_c = acc_c + feats[j] * W[j * 2 + k]
            acc_r = acc_r + feats[j] * W[64 + j * 2 + k]
        pc.append(acc_c)
        pr.append(acc_r + W[128 + k])

    iota = _iota16()

    for sub in range(2):
        nb0 = vb * 16 + sub * 1568
        pltpu.sync_copy(p4.at[pl.ds(nb0 * 8, 12544)], p4b.at[pl.ds(0, 12544)])
        pltpu.sync_copy(p4.at[pl.ds(N * 8 + nb0 * 8, 12544)],
                        p4b.at[pl.ds(12544, 12544)])
        pltpu.sync_copy(p3c.at[pl.ds(nb0 * 8, 12544)],
                        p3b.at[pl.ds(0, 12544)])
        pltpu.sync_copy(p3c.at[pl.ds(N * 8 + nb0 * 8, 12544)],
                        p3b.at[pl.ds(12544, 12544)])
        pltpu.sync_copy(r0.at[pl.ds(nb0 * 2, 3136)], rbuf)
        pltpu.sync_copy(c_f.at[pl.ds(nb0 * 2, 3136)], cbuf)

        def vloop(i, carry):
            nid = i * 16 + iota
            nid2 = nid * 2
            nid8 = nid * 8
            dg = (plsc.load_gather(p3b, [nid8 + 2])
                  + plsc.load_gather(p3b, [nid8 + 12546]))
            for k in range(2):
                s4 = (plsc.load_gather(p4b, [nid8 + k])
                      + plsc.load_gather(p4b, [nid8 + 12544 + k]))
                rv = plsc.load_gather(rbuf, [nid2 + k])
                cv = plsc.load_gather(cbuf, [nid2 + k])
                pre = s4 + dg * pc[k] + rv + pr[k]
                sg = 1.0 / (1.0 + jnp.exp(-pre))
                plsc.store_scatter(ob, [nid2 + k], sg + cv)
            return carry

        lax.fori_loop(0, 98, vloop, 0)
        pltpu.sync_copy(ob, outf.at[pl.ds(nb0 * 2, 3136)])


_phase_d2 = pl.kernel(
    _phase_d2_body,
    out_type=[jax.ShapeDtypeStruct((N * 2,), jnp.float32)],
    mesh=_MESH,
    compiler_params=_DENSE_PARAMS,
    scratch_types=[pltpu.VMEM((NW * 256,), jnp.float32),
                   pltpu.VMEM((NW * 256,), jnp.float32),
                   pltpu.VMEM((25088,), jnp.float32),
                   pltpu.VMEM((25088,), jnp.float32),
                   pltpu.VMEM((3136,), jnp.float32),
                   pltpu.VMEM((3136,), jnp.float32),
                   pltpu.VMEM((3136,), jnp.float32),
                   pltpu.VMEM((144,), jnp.float32)])


def kernel(o, m, c, edge_index_o, edge_index_m, edge_index_c,
           W1_rel, W1_root, b1, W2_rel, W2_root, b2,
           W3_rel, W3_root, b3, W4_rel, W4_root, b4):
    f32 = jnp.float32
    pad6 = jnp.zeros((N, 6), f32)
    o8 = jnp.concatenate([o, pad6], axis=1)
    m8 = jnp.concatenate([m, pad6], axis=1)
    caug8 = jnp.concatenate([c, jnp.ones((N, 1), f32),
                             jnp.zeros((N, 5), f32)], axis=1)
    srco = edge_index_o[0].reshape(NCH, 128)
    dsto = edge_index_o[1].reshape(NCH, 128)
    srcm = edge_index_m[0].reshape(NCH, 128)
    dstm = edge_index_m[1].reshape(NCH, 128)
    srcc = edge_index_c[0].reshape(NCH, 128)
    dstc = edge_index_c[1].reshape(NCH, 128)
    z8 = jnp.zeros((RT, F), f32)

    p1o, p1m = _seg2(o8, m8, srco, srcm, dsto, dstm, z8)
    (p3c,) = _seg1(caug8, srcc, dstc, z8)

    wpkb = jnp.concatenate(
        [W1_rel.reshape(-1), W1_root.reshape(-1), b1,
         W3_rel.reshape(-1), W3_root.reshape(-1), b3,
         W4_rel[:16].reshape(-1), W4_root[:16].reshape(-1),
         jnp.zeros((8,), f32)])
    h1o, h1m, g0, r0 = _phase_b(
        o.reshape(-1), m.reshape(-1), c.reshape(-1),
        p1o.reshape(-1), p1m.reshape(-1), p3c.reshape(-1), wpkb)

    p2o, p2m = _seg2(h1o.reshape(N, 8), h1m.reshape(N, 8),
                     srco, srcm, dsto, dstm, z8)
    (p4,) = _seg1(g0.reshape(N, 8), srcc, dstc, z8)

    wpk1 = jnp.concatenate([W2_rel.reshape(-1), W2_root.reshape(-1), b2])
    pmo, pmm = _phase_d1(p2o.reshape(-1), p2m.reshape(-1), h1o, h1m, wpk1)

    wpk2 = jnp.concatenate(
        [W4_rel[16:].reshape(-1), W4_root[16:].reshape(-1), b4,
         jnp.zeros((14,), f32)])
    (outf,) = _phase_d2(pmo.reshape(-1), pmm.reshape(-1),
                        p4.reshape(-1), p3c.reshape(-1),
                        r0, c.reshape(-1), wpk2)
    return outf.reshape(N, 2)
